# baseline (speedup 1.0000x reference)
import dataclasses

import jax
import jax.numpy as jnp
import numpy as np
from jax import lax
from jax.experimental import pallas as pl
from jax.experimental.pallas import tpu as pltpu
from jax.experimental.pallas import tpu_sc as plsc

EPS = 1e-10
PHI = (1.0 + np.sqrt(5.0)) / 2.0
PHI4 = float(PHI ** 4)
DT = float(PHI ** (-2.0))
MAGIC = 0x5F3759DF

N = 100000
E = 6400000
NW = 32
K = 256
CPW = 784
Ep = NW * K * CPW
PAD = Ep - E
Np = 100096


def _prelude_body(ell_ref, th_ref, x_ref, y_ref):
    l = ell_ref[...]
    t = th_ref[...]
    c = jnp.cos(t)
    s = jnp.sin(t)
    e = jnp.exp(l)
    x_ref[...] = jnp.sign(c) * e * (jnp.abs(c) + EPS)
    y_ref[...] = jnp.sign(s) * e * (jnp.abs(s) + EPS)


def _epilogue_body(ae0, at0, dg0, ae1, at1, dg1, ell_ref, th_ref, out_ref):
    deg = dg0[...] + dg1[...]
    inv = DT / (deg + EPS)
    out_ref[0, :] = ell_ref[...] + (ae0[...] + ae1[...]) * inv
    out_ref[1, :] = th_ref[...] + (at0[...] + at1[...]) * inv


def _rsqrt(v):
    bits = lax.bitcast_convert_type(v, jnp.int32)
    y = lax.bitcast_convert_type(MAGIC - (bits >> 1), jnp.float32)
    y = y * (1.5 - 0.5 * v * y * y)
    y = y * (1.5 - 0.5 * v * y * y)
    y = y * (1.5 - 0.5 * v * y * y)
    return y


def _make_sc_call(n_pad, k, cpw, interpret=False):
    rpt = n_pad // 16
    r = k // 128

    def _sc_body(tx_hbm, ty_hbm, tl_hbm, src_hbm, dst_hbm, zero_hbm,
                 oae0, oat0, odg0, oae1, oat1, odg1,
                 srcv, dstv, sx, sy, sl, dx_, dy_, me, mt, mc,
                 zb, acc_e, acc_t, acc_d, sem_i, sem_g):
        cid = lax.axis_index("c")
        sid = lax.axis_index("s")
        w = cid * 16 + sid
        sl_ = pl.ds(sid * rpt, rpt)

        pltpu.sync_copy(zero_hbm.at[sl_], zb)
        pltpu.sync_copy(zb, acc_e.at[sl_])
        pltpu.sync_copy(zb, acc_t.at[sl_])
        pltpu.sync_copy(zb, acc_d.at[sl_])
        plsc.subcore_barrier()

        @pl.loop(0, cpw)
        def _chunk(ci):
            row0 = (w * cpw + ci) * r
            cp_s = pltpu.async_copy(src_hbm.at[pl.ds(row0, r)], srcv, sem_i)
            cp_d = pltpu.async_copy(dst_hbm.at[pl.ds(row0, r)], dstv, sem_i)
            cp_s.wait()
            cp_d.wait()
            gathers = []
            for j in range(r):
                pj = pl.ds(j * 128, 128)
                gathers.append(pltpu.async_copy(
                    tx_hbm.at[srcv.at[j]], sx.at[pj], sem_g))
                gathers.append(pltpu.async_copy(
                    ty_hbm.at[srcv.at[j]], sy.at[pj], sem_g))
                gathers.append(pltpu.async_copy(
                    tl_hbm.at[srcv.at[j]], sl.at[pj], sem_g))
                gathers.append(pltpu.async_copy(
                    tx_hbm.at[dstv.at[j]], dx_.at[pj], sem_g))
                gathers.append(pltpu.async_copy(
                    ty_hbm.at[dstv.at[j]], dy_.at[pj], sem_g))
            for g in gathers:
                g.wait()

            @pl.loop(0, k, step=16)
            def _c16(r0):
                pv = pl.ds(r0, 16)
                Xsj = sx[pv]
                Ysj = sy[pv]
                elj = sl[pv]
                Xsi = dx_[pv]
                Ysi = dy_[pv]
                aXi = jnp.abs(Xsi)
                aXj = jnp.abs(Xsj)
                xmx = jnp.maximum(aXi, aXj)
                dx = jnp.where(Xsi * Xsj > 0,
                               xmx - jnp.minimum(aXi, aXj) + EPS * xmx,
                               aXi + aXj)
                aYi = jnp.abs(Ysi)
                aYj = jnp.abs(Ysj)
                ymx = jnp.maximum(aYi, aYj)
                dy = jnp.where(Ysi * Ysj > 0,
                               ymx - jnp.minimum(aYi, aYj) + EPS * ymx,
                               aYi + aYj)
                d2 = dx * dx + dy * dy
                coup = jnp.where(d2 < PHI4, _rsqrt(d2), 0.0)
                ii = _rsqrt((Xsi * Xsi + Ysi * Ysi) *
                            (Xsj * Xsj + Ysj * Ysj))
                sji = (Ysj * Xsi - Xsj * Ysi) * ii
                me[pv] = coup * elj
                mt[pv] = coup * sji
                mc[pv] = coup

            for j in range(r):
                pj = pl.ds(j * 128, 128)
                pltpu.sync_copy(me.at[pj], acc_e.at[dstv.at[j]], add=True)
                pltpu.sync_copy(mt.at[pj], acc_t.at[dstv.at[j]], add=True)
                pltpu.sync_copy(mc.at[pj], acc_d.at[dstv.at[j]], add=True)

        plsc.subcore_barrier()

        @pl.when(cid == 0)
        def _dump0():
            pltpu.sync_copy(acc_e.at[sl_], zb)
            pltpu.sync_copy(zb, oae0.at[sl_])
            pltpu.sync_copy(acc_t.at[sl_], zb)
            pltpu.sync_copy(zb, oat0.at[sl_])
            pltpu.sync_copy(acc_d.at[sl_], zb)
            pltpu.sync_copy(zb, odg0.at[sl_])

        @pl.when(cid == 1)
        def _dump1():
            pltpu.sync_copy(acc_e.at[sl_], zb)
            pltpu.sync_copy(zb, oae1.at[sl_])
            pltpu.sync_copy(acc_t.at[sl_], zb)
            pltpu.sync_copy(zb, oat1.at[sl_])
            pltpu.sync_copy(acc_d.at[sl_], zb)
            pltpu.sync_copy(zb, odg1.at[sl_])

    cp = pltpu.CompilerParams(use_tc_tiling_on_sc=False)
    if "needs_layout_passes" in pltpu.CompilerParams.__dataclass_fields__:
        cp = dataclasses.replace(cp, needs_layout_passes=False)

    v1 = jax.ShapeDtypeStruct((n_pad,), jnp.float32)
    return pl.kernel(
        _sc_body,
        out_type=(v1, v1, v1, v1, v1, v1),
        mesh=plsc.VectorSubcoreMesh(core_axis_name="c", subcore_axis_name="s"),
        scratch_types=[
            pltpu.VMEM((r, 128), jnp.int32),
            pltpu.VMEM((r, 128), jnp.int32),
            pltpu.VMEM((k,), jnp.float32),
            pltpu.VMEM((k,), jnp.float32),
            pltpu.VMEM((k,), jnp.float32),
            pltpu.VMEM((k,), jnp.float32),
            pltpu.VMEM((k,), jnp.float32),
            pltpu.VMEM((k,), jnp.float32),
            pltpu.VMEM((k,), jnp.float32),
            pltpu.VMEM((k,), jnp.float32),
            pltpu.VMEM((rpt,), jnp.float32),
            pltpu.VMEM_SHARED((n_pad,), jnp.float32),
            pltpu.VMEM_SHARED((n_pad,), jnp.float32),
            pltpu.VMEM_SHARED((n_pad,), jnp.float32),
            pltpu.SemaphoreType.DMA,
            pltpu.SemaphoreType.DMA,
        ],
        compiler_params=cp,
        interpret=interpret,
    )


_sc_call = _make_sc_call(Np, K, CPW)


def kernel(ell, theta, edge_index):
    idx32 = edge_index.astype(jnp.int32)
    fillv = jnp.full((PAD,), N, jnp.int32)
    src = jnp.concatenate([idx32[0], fillv]).reshape(Ep // 128, 128)
    dst = jnp.concatenate([idx32[1], fillv]).reshape(Ep // 128, 128)
    ellp = jnp.pad(ell, (0, Np - N))
    thp = jnp.pad(theta, (0, Np - N))
    v1 = jax.ShapeDtypeStruct((Np,), jnp.float32)
    tabx, taby = pl.pallas_call(
        _prelude_body, out_shape=(v1, v1))(ellp, thp)
    zeros = jnp.zeros((Np,), jnp.float32)
    ae0, at0, dg0, ae1, at1, dg1 = _sc_call(
        tabx, taby, ellp, src, dst, zeros)
    outp = pl.pallas_call(
        _epilogue_body,
        out_shape=jax.ShapeDtypeStruct((2, Np), jnp.float32),
    )(ae0, at0, dg0, ae1, at1, dg1, ellp, thp)
    return outp[:, :N]
